# baseline (speedup 1.0000x reference)
import jax
import jax.numpy as jnp
from jax import lax
from jax.experimental import pallas as pl
from jax.experimental.pallas import tpu as pltpu
from jax.experimental.pallas import tpu_sc as plsc

N_NODES = 10000
N_EDGES = 320000
D = 128

NC = 2
NS = 16
NW = NC * NS
E_PT = N_EDGES // NW
CHUNK = 80
NCHUNK = E_PT // CHUNK
NB = 4
NGROUP = 31
ROWS_A = 632
ROWS_LAST = N_NODES - ROWS_A * (NS - 1)
DEG_W = 128

_mesh = plsc.VectorSubcoreMesh(core_axis_name="c", subcore_axis_name="s")


def _zero_acc(sid, zeros_hbm, acc):
    row0 = sid * ROWS_A

    @pl.when(sid < NS - 1)
    def _():
        pltpu.sync_copy(zeros_hbm.at[pl.ds(0, ROWS_A)], acc.at[pl.ds(row0, ROWS_A)])

    @pl.when(sid == NS - 1)
    def _():
        pltpu.sync_copy(zeros_hbm.at[pl.ds(0, ROWS_LAST)],
                        acc.at[pl.ds(row0, ROWS_LAST)])


def _copy_out(sid, cid, acc, out_hbm):
    row0 = sid * ROWS_A

    @pl.when(sid < NS - 1)
    def _():
        pltpu.sync_copy(acc.at[pl.ds(row0, ROWS_A)],
                        out_hbm.at[cid, pl.ds(row0, ROWS_A)])

    @pl.when(sid == NS - 1)
    def _():
        pltpu.sync_copy(acc.at[pl.ds(row0, ROWS_LAST)],
                        out_hbm.at[cid, pl.ds(row0, ROWS_LAST)])


DCHUNK = 128
DNCHUNK = 80
D_E_PT = DCHUNK * DNCHUNK
DNGROUP = DNCHUNK // NB
PAD_ROW = N_NODES
NPD = N_NODES + 8


def _deg_body(dst_hbm, ones_hbm, zeros_hbm, out_hbm,
              didx2, ones_v, acc, s0, s1, s2, s3):
    cid = lax.axis_index("c")
    sid = lax.axis_index("s")
    wid = cid * NS + sid
    sems = [s0, s1, s2, s3]
    row0 = sid * ROWS_A
    last = NPD - ROWS_A * (NS - 1)

    pltpu.sync_copy(ones_hbm, ones_v)

    @pl.when(sid < NS - 1)
    def _():
        pltpu.sync_copy(zeros_hbm.at[pl.ds(0, ROWS_A)], acc.at[pl.ds(row0, ROWS_A)])

    @pl.when(sid == NS - 1)
    def _():
        pltpu.sync_copy(zeros_hbm.at[pl.ds(0, last)], acc.at[pl.ds(row0, last)])

    plsc.subcore_barrier()
    pltpu.sync_copy(dst_hbm.at[wid], didx2)

    for b in range(NB):
        pltpu.async_copy(ones_v, acc.at[didx2.at[b]], sems[b], add=True)

    def group(g, carry):
        for b in range(NB):
            pltpu.make_async_copy(ones_v, acc.at[didx2.at[0]], sems[b]).wait()
            jn = (g + 1) * NB + b

            @pl.when(jn < DNCHUNK)
            def _():
                pltpu.async_copy(ones_v, acc.at[didx2.at[jn]], sems[b], add=True)

        return carry

    lax.fori_loop(0, DNGROUP, group, None)

    plsc.subcore_barrier()

    @pl.when(sid < NS - 1)
    def _():
        pltpu.sync_copy(acc.at[pl.ds(row0, ROWS_A)],
                        out_hbm.at[cid, pl.ds(row0, ROWS_A)])

    @pl.when(sid == NS - 1)
    def _():
        pltpu.sync_copy(acc.at[pl.ds(row0, last)],
                        out_hbm.at[cid, pl.ds(row0, last)])


_deg_kernel = pl.kernel(
    _deg_body,
    out_type=jax.ShapeDtypeStruct((NC, NPD, DEG_W), jnp.float32),
    mesh=_mesh,
    scratch_types=[
        pltpu.VMEM((DNCHUNK, DCHUNK), jnp.int32),
        pltpu.VMEM((DCHUNK, DEG_W), jnp.float32),
        pltpu.VMEM_SHARED((NPD, DEG_W), jnp.float32),
        pltpu.SemaphoreType.DMA,
        pltpu.SemaphoreType.DMA,
        pltpu.SemaphoreType.DMA,
        pltpu.SemaphoreType.DMA,
    ],
)


def _agg_body(hp_hbm, src_hbm, dst_hbm, zeros_hbm, out_hbm,
              si0, si1, si2, si3, di0, di1, di2, di3,
              r0, r1, r2, r3, acc,
              ia, ib, ic, id_, ga, gb, gc, gd, sa, sb, sc, sd):
    cid = lax.axis_index("c")
    sid = lax.axis_index("s")
    wid = cid * NS + sid
    sidx = [si0, si1, si2, si3]
    didx = [di0, di1, di2, di3]
    rows = [r0, r1, r2, r3]
    semi = [ia, ib, ic, id_]
    semg = [ga, gb, gc, gd]
    sems = [sa, sb, sc, sd]
    LAST = NGROUP * NB

    base = wid * E_PT

    _zero_acc(sid, zeros_hbm, acc)
    plsc.subcore_barrier()

    for b in range(NB):
        pltpu.async_copy(src_hbm.at[pl.ds(base + b * CHUNK, CHUNK)], sidx[b], semi[b])
        pltpu.async_copy(dst_hbm.at[pl.ds(base + b * CHUNK, CHUNK)], didx[b], semi[b])
    for b in range(NB):
        pltpu.make_async_copy(src_hbm.at[pl.ds(0, CHUNK)], sidx[b], semi[b]).wait()
        pltpu.make_async_copy(dst_hbm.at[pl.ds(0, CHUNK)], didx[b], semi[b]).wait()
        pltpu.async_copy(hp_hbm.at[sidx[b]], rows[b], semg[b])

    def group(g, carry):
        for b in range(NB):
            pltpu.make_async_copy(hp_hbm.at[sidx[0]], rows[b], semg[b]).wait()
            pltpu.async_copy(rows[b], acc.at[didx[b]], sems[b], add=True)
        for b in range(NB):
            jn = (g + 1) * NB + b
            pltpu.make_async_copy(rows[b], acc.at[didx[0]], sems[b]).wait()

            @pl.when(jn < LAST)
            def _():
                off = base + jn * CHUNK
                pltpu.async_copy(src_hbm.at[pl.ds(off, CHUNK)], sidx[b], semi[b])
                pltpu.async_copy(dst_hbm.at[pl.ds(off, CHUNK)], didx[b], semi[b])
                pltpu.make_async_copy(src_hbm.at[pl.ds(0, CHUNK)], sidx[b], semi[b]).wait()
                pltpu.make_async_copy(dst_hbm.at[pl.ds(0, CHUNK)], didx[b], semi[b]).wait()
                pltpu.async_copy(hp_hbm.at[sidx[b]], rows[b], semg[b])

        return carry

    lax.fori_loop(0, NGROUP, group, None)

    off = base + LAST * CHUNK
    pltpu.sync_copy(src_hbm.at[pl.ds(off, CHUNK)], si0)
    pltpu.sync_copy(dst_hbm.at[pl.ds(off, CHUNK)], di0)
    pltpu.async_copy(hp_hbm.at[si0], r0, ga).wait()
    pltpu.sync_copy(r0, acc.at[di0], add=True)

    plsc.subcore_barrier()
    _copy_out(sid, cid, acc, out_hbm)


_agg_kernel = pl.kernel(
    _agg_body,
    out_type=jax.ShapeDtypeStruct((NC, N_NODES, D), jnp.float32),
    mesh=_mesh,
    scratch_types=(
        [pltpu.VMEM((CHUNK,), jnp.int32)] * 8
        + [pltpu.VMEM((CHUNK, D), jnp.float32)] * 4
        + [pltpu.VMEM_SHARED((N_NODES, D), jnp.float32)]
        + [pltpu.SemaphoreType.DMA] * 12
    ),
)


def _tc0_body(x_ref, w_ref, h_ref):
    h_ref[...] = jnp.dot(x_ref[...], w_ref[...], preferred_element_type=jnp.float32)


def _tc1_body(degp_ref, h_ref, dis_ref, hp_ref):
    deg = degp_ref[0, 0:N_NODES, 0:1] + degp_ref[1, 0:N_NODES, 0:1] + 1.0
    dis = lax.rsqrt(deg)
    dis_ref[...] = dis
    hp_ref[...] = h_ref[...] * dis


def _tc2_body(yp_ref, hp_ref, dis_ref, b_ref, w_ref, out_ref):
    dis = dis_ref[...]
    s = (yp_ref[0] + yp_ref[1] + hp_ref[...]) * dis + b_ref[...]
    h = jnp.maximum(s, 0.0)
    out_ref[...] = jnp.dot(h, w_ref[...], preferred_element_type=jnp.float32) * dis


def _tc3_body(yp_ref, hp_ref, dis_ref, b_ref, wf_ref, bf_ref, out_ref):
    s = (yp_ref[0] + yp_ref[1] + hp_ref[...]) * dis_ref[...] + b_ref[...]
    h = jnp.maximum(s, 0.0)
    out_ref[...] = jnp.dot(h, wf_ref[...], preferred_element_type=jnp.float32) + bf_ref[...]


_tc0 = pl.pallas_call(
    _tc0_body,
    out_shape=jax.ShapeDtypeStruct((N_NODES, D), jnp.float32),
)

_tc1 = pl.pallas_call(
    _tc1_body,
    out_shape=(jax.ShapeDtypeStruct((N_NODES, 1), jnp.float32),
               jax.ShapeDtypeStruct((N_NODES, D), jnp.float32)),
)

_tc2 = pl.pallas_call(
    _tc2_body,
    out_shape=jax.ShapeDtypeStruct((N_NODES, D), jnp.float32),
)

_tc3 = pl.pallas_call(
    _tc3_body,
    out_shape=jax.ShapeDtypeStruct((N_NODES, 8), jnp.float32),
)


@jax.jit
def kernel(x, edge_index, W1, b1, W2, b2, Wf, bf):
    src = edge_index[0].astype(jnp.int32)
    dst = edge_index[1].astype(jnp.int32)
    x = x.astype(jnp.float32)

    dpad = jnp.full((NW, D_E_PT - E_PT), PAD_ROW, jnp.int32)
    dstp_d = jnp.concatenate([dst.reshape(NW, E_PT), dpad], axis=1).reshape(
        NW, DNCHUNK, DCHUNK)

    ones_rows = jnp.ones((DCHUNK, DEG_W), jnp.float32)
    zeros_rows = jnp.zeros((ROWS_A, D), jnp.float32)

    h1 = _tc0(x, W1)
    degp = _deg_kernel(dstp_d, ones_rows, zeros_rows)
    dis, hp1 = _tc1(degp, h1)

    yp1 = _agg_kernel(hp1, src, dst, zeros_rows)
    hp2 = _tc2(yp1, hp1, dis, b1.reshape(1, D), W2)

    yp2 = _agg_kernel(hp2, src, dst, zeros_rows)
    wf_pad = jnp.zeros((D, 8), jnp.float32).at[:, :2].set(Wf)
    bf_pad = jnp.zeros((1, 8), jnp.float32).at[0, :2].set(bf)
    out = _tc3(yp2, hp2, dis, b2.reshape(1, D), wf_pad, bf_pad)
    return out[:, :2]


# speedup vs baseline: 1.0098x; 1.0098x over previous
import jax
import jax.numpy as jnp
from jax import lax
from jax.experimental import pallas as pl
from jax.experimental.pallas import tpu as pltpu
from jax.experimental.pallas import tpu_sc as plsc

N_NODES = 10000
N_EDGES = 320000
D = 128

NC = 2
NS = 16
NW = NC * NS
E_PT = N_EDGES // NW
CHUNK = 80
NCHUNK = E_PT // CHUNK
NB = 4
NGROUP = 31
ROWS_A = 632
ROWS_LAST = N_NODES - ROWS_A * (NS - 1)
DEG_W = 128

_mesh = plsc.VectorSubcoreMesh(core_axis_name="c", subcore_axis_name="s")


def _zero_acc(sid, zeros_hbm, acc):
    row0 = sid * ROWS_A

    @pl.when(sid < NS - 1)
    def _():
        pltpu.sync_copy(zeros_hbm.at[pl.ds(0, ROWS_A)], acc.at[pl.ds(row0, ROWS_A)])

    @pl.when(sid == NS - 1)
    def _():
        pltpu.sync_copy(zeros_hbm.at[pl.ds(0, ROWS_LAST)],
                        acc.at[pl.ds(row0, ROWS_LAST)])


def _copy_out(sid, cid, acc, out_hbm):
    row0 = sid * ROWS_A

    @pl.when(sid < NS - 1)
    def _():
        pltpu.sync_copy(acc.at[pl.ds(row0, ROWS_A)],
                        out_hbm.at[cid, pl.ds(row0, ROWS_A)])

    @pl.when(sid == NS - 1)
    def _():
        pltpu.sync_copy(acc.at[pl.ds(row0, ROWS_LAST)],
                        out_hbm.at[cid, pl.ds(row0, ROWS_LAST)])


DCHUNK = 128
DNCHUNK = 80
D_E_PT = DCHUNK * DNCHUNK
DNGROUP = DNCHUNK // NB
PAD_ROW = N_NODES
NPD = N_NODES + 8


def _deg_body(dst_hbm, ones_hbm, zeros_hbm, out_hbm,
              didx2, ones_v, acc, s0, s1, s2, s3):
    cid = lax.axis_index("c")
    sid = lax.axis_index("s")
    wid = cid * NS + sid
    sems = [s0, s1, s2, s3]
    row0 = sid * ROWS_A
    last = NPD - ROWS_A * (NS - 1)

    pltpu.sync_copy(ones_hbm, ones_v)

    @pl.when(sid < NS - 1)
    def _():
        pltpu.sync_copy(zeros_hbm.at[pl.ds(0, ROWS_A)], acc.at[pl.ds(row0, ROWS_A)])

    @pl.when(sid == NS - 1)
    def _():
        pltpu.sync_copy(zeros_hbm.at[pl.ds(0, last)], acc.at[pl.ds(row0, last)])

    plsc.subcore_barrier()
    pltpu.sync_copy(dst_hbm.at[wid], didx2)

    for b in range(NB):
        pltpu.async_copy(ones_v, acc.at[didx2.at[b]], sems[b], add=True)

    def group(g, carry):
        for b in range(NB):
            pltpu.make_async_copy(ones_v, acc.at[didx2.at[0]], sems[b]).wait()
            jn = (g + 1) * NB + b

            @pl.when(jn < DNCHUNK)
            def _():
                pltpu.async_copy(ones_v, acc.at[didx2.at[jn]], sems[b], add=True)

        return carry

    lax.fori_loop(0, DNGROUP, group, None)

    plsc.subcore_barrier()

    @pl.when(sid < NS - 1)
    def _():
        pltpu.sync_copy(acc.at[pl.ds(row0, ROWS_A)],
                        out_hbm.at[cid, pl.ds(row0, ROWS_A)])

    @pl.when(sid == NS - 1)
    def _():
        pltpu.sync_copy(acc.at[pl.ds(row0, last)],
                        out_hbm.at[cid, pl.ds(row0, last)])


_deg_kernel = pl.kernel(
    _deg_body,
    out_type=jax.ShapeDtypeStruct((NC, NPD, DEG_W), jnp.float32),
    mesh=_mesh,
    scratch_types=[
        pltpu.VMEM((DNCHUNK, DCHUNK), jnp.int32),
        pltpu.VMEM((DCHUNK, DEG_W), jnp.float32),
        pltpu.VMEM_SHARED((NPD, DEG_W), jnp.float32),
        pltpu.SemaphoreType.DMA,
        pltpu.SemaphoreType.DMA,
        pltpu.SemaphoreType.DMA,
        pltpu.SemaphoreType.DMA,
    ],
)


def _agg_body(hp_hbm, src_hbm, dst_hbm, zeros_hbm, out_hbm,
              si0, si1, si2, si3, di0, di1, di2, di3,
              r0, r1, r2, r3, acc,
              ia, ib, ic, id_, ga, gb, gc, gd, sa, sb, sc, sd):
    cid = lax.axis_index("c")
    sid = lax.axis_index("s")
    wid = cid * NS + sid
    sidx = [si0, si1, si2, si3]
    didx = [di0, di1, di2, di3]
    rows = [r0, r1, r2, r3]
    semi = [ia, ib, ic, id_]
    semg = [ga, gb, gc, gd]
    sems = [sa, sb, sc, sd]
    LAST = NGROUP * NB

    base = wid * E_PT

    _zero_acc(sid, zeros_hbm, acc)
    plsc.subcore_barrier()

    for b in range(NB):
        pltpu.async_copy(src_hbm.at[pl.ds(base + b * CHUNK, CHUNK)], sidx[b], semi[b])
        pltpu.async_copy(dst_hbm.at[pl.ds(base + b * CHUNK, CHUNK)], didx[b], semi[b])
    for b in range(NB):
        pltpu.make_async_copy(src_hbm.at[pl.ds(0, CHUNK)], sidx[b], semi[b]).wait()
        pltpu.make_async_copy(dst_hbm.at[pl.ds(0, CHUNK)], didx[b], semi[b]).wait()
        pltpu.async_copy(hp_hbm.at[sidx[b]], rows[b], semg[b])

    def group(g, carry):
        for b in range(NB):
            pltpu.make_async_copy(hp_hbm.at[sidx[0]], rows[b], semg[b]).wait()
            pltpu.async_copy(rows[b], acc.at[didx[b]], sems[b], add=True)
        for b in range(NB):
            jn = (g + 1) * NB + b
            pltpu.make_async_copy(rows[b], acc.at[didx[0]], sems[b]).wait()

            @pl.when(jn < LAST)
            def _():
                off = base + jn * CHUNK
                pltpu.async_copy(src_hbm.at[pl.ds(off, CHUNK)], sidx[b], semi[b])
                pltpu.async_copy(dst_hbm.at[pl.ds(off, CHUNK)], didx[b], semi[b])
                pltpu.make_async_copy(src_hbm.at[pl.ds(0, CHUNK)], sidx[b], semi[b]).wait()
                pltpu.make_async_copy(dst_hbm.at[pl.ds(0, CHUNK)], didx[b], semi[b]).wait()
                pltpu.async_copy(hp_hbm.at[sidx[b]], rows[b], semg[b])

        return carry

    lax.fori_loop(0, NGROUP, group, None)

    off = base + LAST * CHUNK
    pltpu.sync_copy(src_hbm.at[pl.ds(off, CHUNK)], si0)
    pltpu.sync_copy(dst_hbm.at[pl.ds(off, CHUNK)], di0)
    pltpu.async_copy(hp_hbm.at[si0], r0, ga).wait()
    pltpu.sync_copy(r0, acc.at[di0], add=True)

    plsc.subcore_barrier()
    _copy_out(sid, cid, acc, out_hbm)


_agg_kernel = pl.kernel(
    _agg_body,
    out_type=jax.ShapeDtypeStruct((NC, N_NODES, D), jnp.float32),
    mesh=_mesh,
    scratch_types=(
        [pltpu.VMEM((CHUNK,), jnp.int32)] * 8
        + [pltpu.VMEM((CHUNK, D), jnp.float32)] * 4
        + [pltpu.VMEM_SHARED((N_NODES, D), jnp.float32)]
        + [pltpu.SemaphoreType.DMA] * 12
    ),
)


def _tc1_body(degp_ref, x_ref, w_ref, dis_ref, hp_ref):
    deg = degp_ref[0, 0:N_NODES, 0:1] + degp_ref[1, 0:N_NODES, 0:1] + 1.0
    dis = lax.rsqrt(deg)
    dis_ref[...] = dis
    h = jnp.dot(x_ref[...], w_ref[...], preferred_element_type=jnp.float32)
    hp_ref[...] = h * dis


def _tc2_body(yp_ref, hp_ref, dis_ref, b_ref, w_ref, out_ref):
    dis = dis_ref[...]
    s = (yp_ref[0] + yp_ref[1] + hp_ref[...]) * dis + b_ref[...]
    h = jnp.maximum(s, 0.0)
    out_ref[...] = jnp.dot(h, w_ref[...], preferred_element_type=jnp.float32) * dis


def _tc3_body(yp_ref, hp_ref, dis_ref, b_ref, wf_ref, bf_ref, out_ref):
    s = (yp_ref[0] + yp_ref[1] + hp_ref[...]) * dis_ref[...] + b_ref[...]
    h = jnp.maximum(s, 0.0)
    out_ref[...] = jnp.dot(h, wf_ref[...], preferred_element_type=jnp.float32) + bf_ref[...]


_tc1 = pl.pallas_call(
    _tc1_body,
    out_shape=(jax.ShapeDtypeStruct((N_NODES, 1), jnp.float32),
               jax.ShapeDtypeStruct((N_NODES, D), jnp.float32)),
)

_tc2 = pl.pallas_call(
    _tc2_body,
    out_shape=jax.ShapeDtypeStruct((N_NODES, D), jnp.float32),
)

_tc3 = pl.pallas_call(
    _tc3_body,
    out_shape=jax.ShapeDtypeStruct((N_NODES, 8), jnp.float32),
)


@jax.jit
def kernel(x, edge_index, W1, b1, W2, b2, Wf, bf):
    src = edge_index[0].astype(jnp.int32)
    dst = edge_index[1].astype(jnp.int32)
    x = x.astype(jnp.float32)

    dpad = PAD_ROW + (jnp.arange(D_E_PT - E_PT, dtype=jnp.int32) % 8)
    dpad = jnp.broadcast_to(dpad, (NW, D_E_PT - E_PT))
    dstp_d = jnp.concatenate([dst.reshape(NW, E_PT), dpad], axis=1).reshape(
        NW, DNCHUNK, DCHUNK)

    ones_rows = jnp.ones((DCHUNK, DEG_W), jnp.float32)
    zeros_rows = jnp.zeros((ROWS_A, D), jnp.float32)

    degp = _deg_kernel(dstp_d, ones_rows, zeros_rows)
    dis, hp1 = _tc1(degp, x, W1)

    yp1 = _agg_kernel(hp1, src, dst, zeros_rows)
    hp2 = _tc2(yp1, hp1, dis, b1.reshape(1, D), W2)

    yp2 = _agg_kernel(hp2, src, dst, zeros_rows)
    wf_pad = jnp.zeros((D, 8), jnp.float32).at[:, :2].set(Wf)
    bf_pad = jnp.zeros((1, 8), jnp.float32).at[0, :2].set(bf)
    out = _tc3(yp2, hp2, dis, b2.reshape(1, D), wf_pad, bf_pad)
    return out[:, :2]


# speedup vs baseline: 1.0116x; 1.0019x over previous
import jax
import jax.numpy as jnp
from jax import lax
from jax.experimental import pallas as pl
from jax.experimental.pallas import tpu as pltpu
from jax.experimental.pallas import tpu_sc as plsc

N_NODES = 10000
N_EDGES = 320000
D = 128

NC = 2
NS = 16
NW = NC * NS
E_PT = N_EDGES // NW
CHUNK = 80
NCHUNK = E_PT // CHUNK
NB = 4
NGROUP = 31
ROWS_A = 632
ROWS_LAST = N_NODES - ROWS_A * (NS - 1)
DEG_W = 128

_mesh = plsc.VectorSubcoreMesh(core_axis_name="c", subcore_axis_name="s",
                               num_cores=NC, num_subcores=NS)


def _zero_acc(sid, zeros_hbm, acc):
    row0 = sid * ROWS_A

    @pl.when(sid < NS - 1)
    def _():
        pltpu.sync_copy(zeros_hbm.at[pl.ds(0, ROWS_A)], acc.at[pl.ds(row0, ROWS_A)])

    @pl.when(sid == NS - 1)
    def _():
        pltpu.sync_copy(zeros_hbm.at[pl.ds(0, ROWS_LAST)],
                        acc.at[pl.ds(row0, ROWS_LAST)])


def _copy_out(sid, cid, acc, out_hbm):
    row0 = sid * ROWS_A

    @pl.when(sid < NS - 1)
    def _():
        pltpu.sync_copy(acc.at[pl.ds(row0, ROWS_A)],
                        out_hbm.at[cid, pl.ds(row0, ROWS_A)])

    @pl.when(sid == NS - 1)
    def _():
        pltpu.sync_copy(acc.at[pl.ds(row0, ROWS_LAST)],
                        out_hbm.at[cid, pl.ds(row0, ROWS_LAST)])


DCHUNK = 128
DNCHUNK = 80
D_E_PT = DCHUNK * DNCHUNK
DNGROUP = DNCHUNK // NB
PAD_ROW = N_NODES
NPD = N_NODES + 8


def _deg_body(dst_hbm, ones_hbm, zeros_hbm, out_hbm,
              didx2, ones_v, acc, s0, s1, s2, s3):
    cid = lax.axis_index("c")
    sid = lax.axis_index("s")
    wid = cid * NS + sid
    sems = [s0, s1, s2, s3]
    row0 = sid * ROWS_A
    last = NPD - ROWS_A * (NS - 1)

    pltpu.sync_copy(ones_hbm, ones_v)

    @pl.when(sid < NS - 1)
    def _():
        pltpu.sync_copy(zeros_hbm.at[pl.ds(0, ROWS_A)], acc.at[pl.ds(row0, ROWS_A)])

    @pl.when(sid == NS - 1)
    def _():
        pltpu.sync_copy(zeros_hbm.at[pl.ds(0, last)], acc.at[pl.ds(row0, last)])

    plsc.subcore_barrier()
    pltpu.sync_copy(dst_hbm.at[wid], didx2)

    for b in range(NB):
        pltpu.async_copy(ones_v, acc.at[didx2.at[b]], sems[b], add=True)

    def group(g, carry):
        for b in range(NB):
            pltpu.make_async_copy(ones_v, acc.at[didx2.at[0]], sems[b]).wait()
            jn = (g + 1) * NB + b

            @pl.when(jn < DNCHUNK)
            def _():
                pltpu.async_copy(ones_v, acc.at[didx2.at[jn]], sems[b], add=True)

        return carry

    lax.fori_loop(0, DNGROUP, group, None)

    plsc.subcore_barrier()

    @pl.when(sid < NS - 1)
    def _():
        pltpu.sync_copy(acc.at[pl.ds(row0, ROWS_A)],
                        out_hbm.at[cid, pl.ds(row0, ROWS_A)])

    @pl.when(sid == NS - 1)
    def _():
        pltpu.sync_copy(acc.at[pl.ds(row0, last)],
                        out_hbm.at[cid, pl.ds(row0, last)])


_deg_kernel = pl.kernel(
    _deg_body,
    out_type=jax.ShapeDtypeStruct((NC, NPD, DEG_W), jnp.float32),
    mesh=_mesh,
    scratch_types=[
        pltpu.VMEM((DNCHUNK, DCHUNK), jnp.int32),
        pltpu.VMEM((DCHUNK, DEG_W), jnp.float32),
        pltpu.VMEM_SHARED((NPD, DEG_W), jnp.float32),
        pltpu.SemaphoreType.DMA,
        pltpu.SemaphoreType.DMA,
        pltpu.SemaphoreType.DMA,
        pltpu.SemaphoreType.DMA,
    ],
)


def _agg_body(hp_hbm, src_hbm, dst_hbm, zeros_hbm, out_hbm,
              si0, si1, si2, si3, di0, di1, di2, di3,
              r0, r1, r2, r3, acc,
              ia, ib, ic, id_, ga, gb, gc, gd, sa, sb, sc, sd):
    cid = lax.axis_index("c")
    sid = lax.axis_index("s")
    wid = cid * NS + sid
    sidx = [si0, si1, si2, si3]
    didx = [di0, di1, di2, di3]
    rows = [r0, r1, r2, r3]
    semi = [ia, ib, ic, id_]
    semg = [ga, gb, gc, gd]
    sems = [sa, sb, sc, sd]
    LAST = NGROUP * NB

    base = wid * E_PT

    _zero_acc(sid, zeros_hbm, acc)
    plsc.subcore_barrier()

    for b in range(NB):
        pltpu.async_copy(src_hbm.at[pl.ds(base + b * CHUNK, CHUNK)], sidx[b], semi[b])
        pltpu.async_copy(dst_hbm.at[pl.ds(base + b * CHUNK, CHUNK)], didx[b], semi[b])
    for b in range(NB):
        pltpu.make_async_copy(src_hbm.at[pl.ds(0, CHUNK)], sidx[b], semi[b]).wait()
        pltpu.make_async_copy(dst_hbm.at[pl.ds(0, CHUNK)], didx[b], semi[b]).wait()
        pltpu.async_copy(hp_hbm.at[sidx[b]], rows[b], semg[b])

    def group(g, carry):
        for b in range(NB):
            pltpu.make_async_copy(hp_hbm.at[sidx[0]], rows[b], semg[b]).wait()
            pltpu.async_copy(rows[b], acc.at[didx[b]], sems[b], add=True)
        for b in range(NB):
            jn = (g + 1) * NB + b
            pltpu.make_async_copy(rows[b], acc.at[didx[0]], sems[b]).wait()

            @pl.when(jn < LAST)
            def _():
                off = base + jn * CHUNK
                pltpu.async_copy(src_hbm.at[pl.ds(off, CHUNK)], sidx[b], semi[b])
                pltpu.async_copy(dst_hbm.at[pl.ds(off, CHUNK)], didx[b], semi[b])
                pltpu.make_async_copy(src_hbm.at[pl.ds(0, CHUNK)], sidx[b], semi[b]).wait()
                pltpu.make_async_copy(dst_hbm.at[pl.ds(0, CHUNK)], didx[b], semi[b]).wait()
                pltpu.async_copy(hp_hbm.at[sidx[b]], rows[b], semg[b])

        return carry

    lax.fori_loop(0, NGROUP, group, None)

    off = base + LAST * CHUNK
    pltpu.sync_copy(src_hbm.at[pl.ds(off, CHUNK)], si0)
    pltpu.sync_copy(dst_hbm.at[pl.ds(off, CHUNK)], di0)
    pltpu.async_copy(hp_hbm.at[si0], r0, ga).wait()
    pltpu.sync_copy(r0, acc.at[di0], add=True)

    plsc.subcore_barrier()
    _copy_out(sid, cid, acc, out_hbm)


_agg_kernel = pl.kernel(
    _agg_body,
    out_type=jax.ShapeDtypeStruct((NC, N_NODES, D), jnp.float32),
    mesh=_mesh,
    scratch_types=(
        [pltpu.VMEM((CHUNK,), jnp.int32)] * 8
        + [pltpu.VMEM((CHUNK, D), jnp.float32)] * 4
        + [pltpu.VMEM_SHARED((N_NODES, D), jnp.float32)]
        + [pltpu.SemaphoreType.DMA] * 12
    ),
)


def _tc1_body(degp_ref, x_ref, w_ref, dis_ref, hp_ref):
    deg = degp_ref[0, 0:N_NODES, 0:1] + degp_ref[1, 0:N_NODES, 0:1] + 1.0
    dis = lax.rsqrt(deg)
    dis_ref[...] = dis
    h = jnp.dot(x_ref[...], w_ref[...], preferred_element_type=jnp.float32)
    hp_ref[...] = h * dis


def _tc2_body(yp_ref, hp_ref, dis_ref, b_ref, w_ref, out_ref):
    dis = dis_ref[...]
    s = (yp_ref[0] + yp_ref[1] + hp_ref[...]) * dis + b_ref[...]
    h = jnp.maximum(s, 0.0)
    out_ref[...] = jnp.dot(h, w_ref[...], preferred_element_type=jnp.float32) * dis


def _tc3_body(yp_ref, hp_ref, dis_ref, b_ref, wf_ref, bf_ref, out_ref):
    s = (yp_ref[0] + yp_ref[1] + hp_ref[...]) * dis_ref[...] + b_ref[...]
    h = jnp.maximum(s, 0.0)
    out_ref[...] = jnp.dot(h, wf_ref[...], preferred_element_type=jnp.float32) + bf_ref[...]


_tc1 = pl.pallas_call(
    _tc1_body,
    out_shape=(jax.ShapeDtypeStruct((N_NODES, 1), jnp.float32),
               jax.ShapeDtypeStruct((N_NODES, D), jnp.float32)),
)

_tc2 = pl.pallas_call(
    _tc2_body,
    out_shape=jax.ShapeDtypeStruct((N_NODES, D), jnp.float32),
)

_tc3 = pl.pallas_call(
    _tc3_body,
    out_shape=jax.ShapeDtypeStruct((N_NODES, 8), jnp.float32),
)


@jax.jit
def kernel(x, edge_index, W1, b1, W2, b2, Wf, bf):
    src = edge_index[0].astype(jnp.int32)
    dst = edge_index[1].astype(jnp.int32)
    x = x.astype(jnp.float32)

    dpad = PAD_ROW + (jnp.arange(D_E_PT - E_PT, dtype=jnp.int32) % 8)
    dpad = jnp.broadcast_to(dpad, (NW, D_E_PT - E_PT))
    dstp_d = jnp.concatenate([dst.reshape(NW, E_PT), dpad], axis=1).reshape(
        NW, DNCHUNK, DCHUNK)

    ones_rows = jnp.ones((DCHUNK, DEG_W), jnp.float32)
    zeros_rows = jnp.zeros((ROWS_A, D), jnp.float32)

    degp = _deg_kernel(dstp_d, ones_rows, zeros_rows)
    dis, hp1 = _tc1(degp, x, W1)

    yp1 = _agg_kernel(hp1, src, dst, zeros_rows)
    hp2 = _tc2(yp1, hp1, dis, b1.reshape(1, D), W2)

    yp2 = _agg_kernel(hp2, src, dst, zeros_rows)
    wf_pad = jnp.zeros((D, 8), jnp.float32).at[:, :2].set(Wf)
    bf_pad = jnp.zeros((1, 8), jnp.float32).at[0, :2].set(bf)
    out = _tc3(yp2, hp2, dis, b2.reshape(1, D), wf_pad, bf_pad)
    return out[:, :2]
